# baseline (speedup 1.0000x reference)
import jax
import jax.numpy as jnp
from jax.experimental import pallas as pl
from jax.experimental.pallas import tpu as pltpu
from jax.experimental.pallas import tpu_sc as plsc

N_TOKENS = 32768
EMBED = 768
N_EXP = 5
NUM_HEADS = 12
D_K = EMBED // NUM_HEADS

GATHER_WIN = 128
TOKEN_BLK = 1024


N_SUBCORES = 32
HALF = 64


def _sc_gather(x, idx2d):
    mesh = plsc.VectorSubcoreMesh(core_axis_name="core", subcore_axis_name="subcore")
    width = x.shape[1]
    n = idx2d.shape[1]
    chunk = n // N_SUBCORES

    @pl.kernel(
        out_type=jax.ShapeDtypeStruct((n, width), x.dtype),
        mesh=mesh,
        scratch_types=[
            pltpu.VMEM((1, chunk), jnp.int32),
            pltpu.VMEM((HALF, width), x.dtype),
            pltpu.VMEM((HALF, width), x.dtype),
            pltpu.SemaphoreType.DMA,
            pltpu.SemaphoreType.DMA,
            pltpu.SemaphoreType.DMA,
        ],
    )
    def k(x_hbm, i_hbm, o_hbm, idx_buf, buf0, buf1, sem_i, sem0, sem1):
        core = jax.lax.axis_index("core")
        sub = jax.lax.axis_index("subcore")
        base = (core * 16 + sub) * chunk
        pltpu.make_async_copy(
            i_hbm.at[pl.ds(0, 1), pl.ds(base, chunk)], idx_buf, sem_i
        ).start()
        pltpu.make_async_copy(
            i_hbm.at[pl.ds(0, 1), pl.ds(base, chunk)], idx_buf, sem_i
        ).wait()

        @pl.loop(0, chunk // (2 * HALF))
        def _(s):
            r0 = 2 * s * HALF
            r1 = r0 + HALF
            pltpu.sync_copy(x_hbm.at[idx_buf.at[0, pl.ds(r0, HALF)]], buf0)
            wb0 = pltpu.make_async_copy(
                buf0, o_hbm.at[pl.ds(base + r0, HALF), :], sem0)
            wb0.start()
            pltpu.sync_copy(x_hbm.at[idx_buf.at[0, pl.ds(r1, HALF)]], buf1)
            wb1 = pltpu.make_async_copy(
                buf1, o_hbm.at[pl.ds(base + r1, HALF), :], sem1)
            wb1.start()
            wb0.wait()
            wb1.wait()

    return k(x, idx2d)


def _tc_moe_body(m_ref, g_ref, w_ref, b_ref, o_ref):
    g = g_ref[...].astype(jnp.bfloat16)
    m = m_ref[...]
    prio = jax.lax.broadcasted_iota(jnp.int32, m.shape, 1) + 1
    e = jnp.max(prio * m, axis=1, keepdims=True) - 1
    acc = jnp.zeros((g.shape[0], EMBED), jnp.float32)
    for i in range(N_EXP):
        y = jnp.dot(g, w_ref[i], preferred_element_type=jnp.float32)
        y = y + b_ref[i:i + 1, :]
        acc = jnp.where(e == i, y, acc)
    o_ref[...] = acc


CHUNK_SIZES = (8192, 12288, 12288)


def _tc_moe_chunk(buf, g, masks_c, w_t_bf16, b, tok_off):
    off = tok_off // TOKEN_BLK
    n_blks = g.shape[0] // TOKEN_BLK
    in_specs = [
        pl.BlockSpec((TOKEN_BLK, 8), lambda i: (i, 0)),
        pl.BlockSpec((TOKEN_BLK, EMBED), lambda i: (i, 0)),
        pl.BlockSpec((N_EXP, EMBED, EMBED), lambda i: (0, 0, 0)),
        pl.BlockSpec((N_EXP, EMBED), lambda i: (0, 0)),
    ]
    args = (masks_c, g, w_t_bf16, b)
    kwargs = {}
    body = _tc_moe_body
    if tok_off > 0:
        in_specs = [pl.BlockSpec(memory_space=pltpu.MemorySpace.HBM)] + in_specs
        args = (buf,) + args
        kwargs = {"input_output_aliases": {0: 0}}
        body = lambda buf_ref, *rest: _tc_moe_body(*rest)
    return pl.pallas_call(
        body,
        grid=(n_blks,),
        in_specs=in_specs,
        out_specs=pl.BlockSpec((TOKEN_BLK, EMBED), lambda i: (i + off, 0)),
        out_shape=jax.ShapeDtypeStruct((N_TOKENS, EMBED), jnp.float32),
        **kwargs,
    )(*args)


def kernel(x, index, masks, W, b):
    idx2d = index.astype(jnp.int32).reshape(1, N_TOKENS)
    masks_pad = jnp.zeros((N_TOKENS, 8), jnp.int32).at[:, :N_EXP].set(
        masks.astype(jnp.int32).T)
    w_t = W.transpose(0, 2, 1).astype(jnp.bfloat16)
    out = None
    tok_off = 0
    for sz in CHUNK_SIZES:
        sl = slice(tok_off, tok_off + sz)
        g_c = _sc_gather(x, idx2d[:, sl])
        out = _tc_moe_chunk(out, g_c, masks_pad[sl], w_t, b, tok_off)
        tok_off += sz
    return out.reshape(N_TOKENS, NUM_HEADS, D_K)


# speedup vs baseline: 1.0199x; 1.0199x over previous
import jax
import jax.numpy as jnp
from jax.experimental import pallas as pl
from jax.experimental.pallas import tpu as pltpu
from jax.experimental.pallas import tpu_sc as plsc

N_TOKENS = 32768
EMBED = 768
N_EXP = 5
NUM_HEADS = 12
D_K = EMBED // NUM_HEADS

GATHER_WIN = 128
TOKEN_BLK = 1024


N_SUBCORES = 32
HALF = 64


def _sc_gather(x, idx2d):
    mesh = plsc.VectorSubcoreMesh(core_axis_name="core", subcore_axis_name="subcore")
    width = x.shape[1]
    n = idx2d.shape[1]
    chunk = n // N_SUBCORES

    @pl.kernel(
        out_type=jax.ShapeDtypeStruct((n, width), x.dtype),
        mesh=mesh,
        scratch_types=[
            pltpu.VMEM((1, chunk), jnp.int32),
            pltpu.VMEM((HALF, width), x.dtype),
            pltpu.VMEM((HALF, width), x.dtype),
            pltpu.SemaphoreType.DMA,
            pltpu.SemaphoreType.DMA,
            pltpu.SemaphoreType.DMA,
        ],
    )
    def k(x_hbm, i_hbm, o_hbm, idx_buf, buf0, buf1, sem_i, sem0, sem1):
        core = jax.lax.axis_index("core")
        sub = jax.lax.axis_index("subcore")
        base = (core * 16 + sub) * chunk
        pltpu.make_async_copy(
            i_hbm.at[pl.ds(0, 1), pl.ds(base, chunk)], idx_buf, sem_i
        ).start()
        pltpu.make_async_copy(
            i_hbm.at[pl.ds(0, 1), pl.ds(base, chunk)], idx_buf, sem_i
        ).wait()

        @pl.loop(0, chunk // (2 * HALF))
        def _(s):
            r0 = 2 * s * HALF
            r1 = r0 + HALF
            pltpu.sync_copy(x_hbm.at[idx_buf.at[0, pl.ds(r0, HALF)]], buf0)
            wb0 = pltpu.make_async_copy(
                buf0, o_hbm.at[pl.ds(base + r0, HALF), :], sem0)
            wb0.start()
            pltpu.sync_copy(x_hbm.at[idx_buf.at[0, pl.ds(r1, HALF)]], buf1)
            wb1 = pltpu.make_async_copy(
                buf1, o_hbm.at[pl.ds(base + r1, HALF), :], sem1)
            wb1.start()
            wb0.wait()
            wb1.wait()

    return k(x, idx2d)


def _tc_moe_body(m_ref, g_ref, w_ref, b_ref, o_ref):
    g = g_ref[...].astype(jnp.bfloat16)
    m = m_ref[...]
    prio = jax.lax.broadcasted_iota(jnp.int32, m.shape, 1) + 1
    e = jnp.max(prio * m, axis=1, keepdims=True) - 1
    acc = jnp.zeros((g.shape[0], EMBED), jnp.float32)
    for i in range(N_EXP):
        y = jnp.dot(g, w_ref[i], preferred_element_type=jnp.float32)
        y = y + b_ref[i:i + 1, :]
        acc = jnp.where(e == i, y, acc)
    o_ref[...] = acc


CHUNK_SIZES = (12288, 20480)


def _tc_moe_chunk(buf, g, masks_c, w_t_bf16, b, tok_off):
    off = tok_off // TOKEN_BLK
    n_blks = g.shape[0] // TOKEN_BLK
    in_specs = [
        pl.BlockSpec((TOKEN_BLK, 8), lambda i: (i, 0)),
        pl.BlockSpec((TOKEN_BLK, EMBED), lambda i: (i, 0)),
        pl.BlockSpec((N_EXP, EMBED, EMBED), lambda i: (0, 0, 0)),
        pl.BlockSpec((N_EXP, EMBED), lambda i: (0, 0)),
    ]
    args = (masks_c, g, w_t_bf16, b)
    kwargs = {}
    body = _tc_moe_body
    if tok_off > 0:
        in_specs = [pl.BlockSpec(memory_space=pltpu.MemorySpace.HBM)] + in_specs
        args = (buf,) + args
        kwargs = {"input_output_aliases": {0: 0}}
        body = lambda buf_ref, *rest: _tc_moe_body(*rest)
    return pl.pallas_call(
        body,
        grid=(n_blks,),
        in_specs=in_specs,
        out_specs=pl.BlockSpec((TOKEN_BLK, EMBED), lambda i: (i + off, 0)),
        out_shape=jax.ShapeDtypeStruct((N_TOKENS, EMBED), jnp.float32),
        **kwargs,
    )(*args)


def kernel(x, index, masks, W, b):
    idx2d = index.astype(jnp.int32).reshape(1, N_TOKENS)
    masks_pad = jnp.zeros((N_TOKENS, 8), jnp.int32).at[:, :N_EXP].set(
        masks.astype(jnp.int32).T)
    w_t = W.transpose(0, 2, 1).astype(jnp.bfloat16)
    out = None
    tok_off = 0
    for sz in CHUNK_SIZES:
        sl = slice(tok_off, tok_off + sz)
        g_c = _sc_gather(x, idx2d[:, sl])
        out = _tc_moe_chunk(out, g_c, masks_pad[sl], w_t, b, tok_off)
        tok_off += sz
    return out.reshape(N_TOKENS, NUM_HEADS, D_K)


# speedup vs baseline: 1.0260x; 1.0060x over previous
import jax
import jax.numpy as jnp
from jax.experimental import pallas as pl
from jax.experimental.pallas import tpu as pltpu
from jax.experimental.pallas import tpu_sc as plsc

N_TOKENS = 32768
EMBED = 768
N_EXP = 5
NUM_HEADS = 12
D_K = EMBED // NUM_HEADS

GATHER_WIN = 128
TOKEN_BLK = 1024


N_SUBCORES = 32
HALF = 64


def _sc_gather(x, idx2d):
    mesh = plsc.VectorSubcoreMesh(core_axis_name="core", subcore_axis_name="subcore")
    width = x.shape[1]
    n = idx2d.shape[1]
    chunk = n // N_SUBCORES

    @pl.kernel(
        out_type=jax.ShapeDtypeStruct((n, width), x.dtype),
        mesh=mesh,
        scratch_types=[
            pltpu.VMEM((1, chunk), jnp.int32),
            pltpu.VMEM((HALF, width), x.dtype),
            pltpu.VMEM((HALF, width), x.dtype),
            pltpu.SemaphoreType.DMA,
            pltpu.SemaphoreType.DMA,
            pltpu.SemaphoreType.DMA,
        ],
    )
    def k(x_hbm, i_hbm, o_hbm, idx_buf, buf0, buf1, sem_i, sem0, sem1):
        core = jax.lax.axis_index("core")
        sub = jax.lax.axis_index("subcore")
        base = (core * 16 + sub) * chunk
        pltpu.make_async_copy(
            i_hbm.at[pl.ds(0, 1), pl.ds(base, chunk)], idx_buf, sem_i
        ).start()
        pltpu.make_async_copy(
            i_hbm.at[pl.ds(0, 1), pl.ds(base, chunk)], idx_buf, sem_i
        ).wait()

        @pl.loop(0, chunk // (2 * HALF))
        def _(s):
            r0 = 2 * s * HALF
            r1 = r0 + HALF
            pltpu.sync_copy(x_hbm.at[idx_buf.at[0, pl.ds(r0, HALF)]], buf0)
            wb0 = pltpu.make_async_copy(
                buf0, o_hbm.at[pl.ds(base + r0, HALF), :], sem0)
            wb0.start()
            pltpu.sync_copy(x_hbm.at[idx_buf.at[0, pl.ds(r1, HALF)]], buf1)
            wb1 = pltpu.make_async_copy(
                buf1, o_hbm.at[pl.ds(base + r1, HALF), :], sem1)
            wb1.start()
            wb0.wait()
            wb1.wait()

    return k(x, idx2d)


def _tc_moe_body(m_ref, g_ref, w_ref, b_ref, o_ref):
    g = g_ref[...].astype(jnp.bfloat16)
    m = m_ref[...]
    prio = jax.lax.broadcasted_iota(jnp.int32, m.shape, 1) + 1
    e = jnp.max(prio * m, axis=1, keepdims=True) - 1
    acc = jnp.zeros((g.shape[0], EMBED), jnp.float32)
    for i in range(N_EXP):
        y = jnp.dot(g, w_ref[i], preferred_element_type=jnp.float32)
        y = y + b_ref[i:i + 1, :]
        acc = jnp.where(e == i, y, acc)
    o_ref[...] = acc


CHUNK_SIZES = (16384, 16384)


def _tc_moe_chunk(buf, g, masks_c, w_t_bf16, b, tok_off):
    off = tok_off // TOKEN_BLK
    n_blks = g.shape[0] // TOKEN_BLK
    in_specs = [
        pl.BlockSpec((TOKEN_BLK, 8), lambda i: (i, 0)),
        pl.BlockSpec((TOKEN_BLK, EMBED), lambda i: (i, 0)),
        pl.BlockSpec((N_EXP, EMBED, EMBED), lambda i: (0, 0, 0)),
        pl.BlockSpec((N_EXP, EMBED), lambda i: (0, 0)),
    ]
    args = (masks_c, g, w_t_bf16, b)
    kwargs = {}
    body = _tc_moe_body
    if tok_off > 0:
        in_specs = [pl.BlockSpec(memory_space=pltpu.MemorySpace.HBM)] + in_specs
        args = (buf,) + args
        kwargs = {"input_output_aliases": {0: 0}}
        body = lambda buf_ref, *rest: _tc_moe_body(*rest)
    return pl.pallas_call(
        body,
        grid=(n_blks,),
        in_specs=in_specs,
        out_specs=pl.BlockSpec((TOKEN_BLK, EMBED), lambda i: (i + off, 0)),
        out_shape=jax.ShapeDtypeStruct((N_TOKENS, EMBED), jnp.float32),
        **kwargs,
    )(*args)


def kernel(x, index, masks, W, b):
    idx2d = index.astype(jnp.int32).reshape(1, N_TOKENS)
    masks_pad = jnp.zeros((N_TOKENS, 8), jnp.int32).at[:, :N_EXP].set(
        masks.astype(jnp.int32).T)
    w_t = W.transpose(0, 2, 1).astype(jnp.bfloat16)
    out = None
    tok_off = 0
    for sz in CHUNK_SIZES:
        sl = slice(tok_off, tok_off + sz)
        g_c = _sc_gather(x, idx2d[:, sl])
        out = _tc_moe_chunk(out, g_c, masks_pad[sl], w_t, b, tok_off)
        tok_off += sz
    return out.reshape(N_TOKENS, NUM_HEADS, D_K)
